# baseline (speedup 1.0000x reference)
import jax
import jax.numpy as jnp
from jax.experimental import pallas as pl
from jax.experimental.pallas import tpu as pltpu

DIM = 1024
HEADS = 16
HDIM = DIM // HEADS
SEQ = 2048
BH, BW = 8, 8
WH, WW = 6, 6
EPS = 1e-6
TPB = SEQ // (BH * BW)
ROW = BW * TPB
KVW = WH * ROW
SCALE = 1.0 / (HDIM ** 0.5)
HPG = 8
CW = HPG * HDIM
NPAIR = HEADS // HPG
NEG = -1e9


def _qkv_kernel(x_ref, wq_ref, wk_ref, wv_ref, o_ref, w_s):
    @pl.when(pl.program_id(0) == 0)
    def _():
        w_s[:, :DIM] = wq_ref[...].astype(jnp.bfloat16)
        w_s[:, DIM:2 * DIM] = wk_ref[...].astype(jnp.bfloat16)
        w_s[:, 2 * DIM:] = wv_ref[...].astype(jnp.bfloat16)

    xb = x_ref[...].astype(jnp.bfloat16)
    o_ref[...] = jnp.dot(xb, w_s[...],
                         preferred_element_type=jnp.float32
                         ).astype(jnp.bfloat16)


def _out_kernel(a_ref, wo_ref, o_ref, w_s):
    @pl.when(pl.program_id(0) == 0)
    def _():
        w_s[...] = wo_ref[...].astype(jnp.bfloat16)

    o_ref[...] = jnp.dot(a_ref[...], w_s[...],
                         preferred_element_type=jnp.float32)


def _rowmeans(x2, n):
    i = jax.lax.broadcasted_iota(jnp.int32, (n, n), 0)
    j = jax.lax.broadcasted_iota(jnp.int32, (n, n), 1)
    ms = jnp.where((i // HDIM) == (j // HDIM), 1.0 / HDIM, 0.0
                   ).astype(jnp.bfloat16)
    return jnp.dot(x2.astype(jnp.bfloat16), ms,
                   preferred_element_type=jnp.float32)


def _attn_kernel(q_ref, k_ref, v_ref, gq_ref, gk_ref, o_ref,
                 kn_ref, vx_ref, cb_ref):
    qi = jax.lax.broadcasted_iota(jnp.int32, (ROW, KVW), 0)
    kj = jax.lax.broadcasted_iota(jnp.int32, (ROW, KVW), 1)
    d = (kj // TPB) % BW - qi // TPB
    ok = (d >= -(WW // 2)) & (d <= WW - 1 - WW // 2)
    cb_ref[...] = jnp.where(ok, 0.0, NEG).astype(jnp.float32)

    gq2 = jnp.concatenate([gq_ref[...]] * HPG, axis=1) * SCALE
    gk2 = jnp.concatenate([gk_ref[...]] * HPG, axis=1)

    k = k_ref[...].astype(jnp.float32)
    km = _rowmeans(k * k, CW)
    kn_ref[...] = (k * jax.lax.rsqrt(km + EPS) * gk2).astype(jnp.bfloat16)
    v = v_ref[...]
    one = jnp.ones((SEQ, HDIM), jnp.bfloat16)
    for u in range(HPG):
        vx_ref[:, 2 * u * HDIM:(2 * u + 1) * HDIM] = \
            v[:, u * HDIM:(u + 1) * HDIM]
        vx_ref[:, (2 * u + 1) * HDIM:(2 * u + 2) * HDIM] = one

    for r in range(BH):
        lo = max(r - WH // 2, 0) * ROW
        hi = min(r + WH - WH // 2, BH) * ROW
        w = hi - lo

        q = q_ref[r * ROW:(r + 1) * ROW, :].astype(jnp.float32)
        qm = _rowmeans(q * q, CW)
        qn = (q * jax.lax.rsqrt(qm + EPS) * gq2).astype(jnp.bfloat16)

        for u in range(HPG):
            s = jax.lax.dot_general(
                qn[:, u * HDIM:(u + 1) * HDIM], kn_ref[lo:hi,
                                                       u * HDIM:(u + 1) * HDIM],
                (((1,), (1,)), ((), ())),
                preferred_element_type=jnp.float32)
            e = jnp.exp(s + cb_ref[:, :w]).astype(jnp.bfloat16)
            pv = jnp.dot(e, vx_ref[lo:hi, 2 * u * HDIM:(2 * u + 2) * HDIM],
                         preferred_element_type=jnp.float32)
            o_ref[r * ROW:(r + 1) * ROW, u * HDIM:(u + 1) * HDIM] = (
                pv[:, :HDIM] * (1.0 / pv[:, HDIM:HDIM + 1])
            ).astype(jnp.bfloat16)


def kernel(x, Wq, Wk, Wv, Wo, gq, gk):
    B = x.shape[0]
    x2 = x.reshape(SEQ, DIM)
    gq1 = gq.reshape(1, HDIM)
    gk1 = gk.reshape(1, HDIM)

    qkv = pl.pallas_call(
        _qkv_kernel,
        grid=(SEQ // ROW,),
        in_specs=[
            pl.BlockSpec((ROW, DIM), lambda i: (i, 0)),
            pl.BlockSpec((DIM, DIM), lambda i: (0, 0)),
            pl.BlockSpec((DIM, DIM), lambda i: (0, 0)),
            pl.BlockSpec((DIM, DIM), lambda i: (0, 0)),
        ],
        out_specs=pl.BlockSpec((ROW, 3 * DIM), lambda i: (i, 0)),
        out_shape=jax.ShapeDtypeStruct((SEQ, 3 * DIM), jnp.bfloat16),
        scratch_shapes=[pltpu.VMEM((DIM, 3 * DIM), jnp.bfloat16)],
        compiler_params=pltpu.CompilerParams(
            dimension_semantics=("arbitrary",)),
    )(x2, Wq, Wk, Wv)

    attn = pl.pallas_call(
        _attn_kernel,
        grid=(NPAIR,),
        in_specs=[
            pl.BlockSpec((SEQ, CW), lambda p: (0, p)),
            pl.BlockSpec((SEQ, CW), lambda p: (0, NPAIR + p)),
            pl.BlockSpec((SEQ, CW), lambda p: (0, 2 * NPAIR + p)),
            pl.BlockSpec((1, HDIM), lambda p: (0, 0)),
            pl.BlockSpec((1, HDIM), lambda p: (0, 0)),
        ],
        out_specs=pl.BlockSpec((SEQ, CW), lambda p: (0, p)),
        out_shape=jax.ShapeDtypeStruct((SEQ, DIM), jnp.bfloat16),
        scratch_shapes=[
            pltpu.VMEM((SEQ, CW), jnp.bfloat16),
            pltpu.VMEM((SEQ, 2 * CW), jnp.bfloat16),
            pltpu.VMEM((ROW, KVW), jnp.float32),
        ],
        compiler_params=pltpu.CompilerParams(
            dimension_semantics=("parallel",)),
    )(qkv, qkv, qkv, gq1, gk1)

    out = pl.pallas_call(
        _out_kernel,
        grid=(SEQ // ROW,),
        in_specs=[
            pl.BlockSpec((ROW, DIM), lambda i: (i, 0)),
            pl.BlockSpec((DIM, DIM), lambda i: (0, 0)),
        ],
        out_specs=pl.BlockSpec((ROW, DIM), lambda i: (i, 0)),
        out_shape=jax.ShapeDtypeStruct((SEQ, DIM), jnp.float32),
        scratch_shapes=[pltpu.VMEM((DIM, DIM), jnp.bfloat16)],
        compiler_params=pltpu.CompilerParams(
            dimension_semantics=("arbitrary",)),
    )(attn, Wo)

    return out.reshape(B, SEQ, DIM)


# speedup vs baseline: 1.0518x; 1.0518x over previous
import jax
import jax.numpy as jnp
from jax.experimental import pallas as pl
from jax.experimental.pallas import tpu as pltpu

DIM = 1024
HEADS = 16
HDIM = DIM // HEADS
SEQ = 2048
BH, BW = 8, 8
WH, WW = 6, 6
EPS = 1e-6
TPB = SEQ // (BH * BW)
ROW = BW * TPB
KVW = WH * ROW
SCALE = 1.0 / (HDIM ** 0.5)
HPG = 4
CW = HPG * HDIM
NPAIR = HEADS // HPG
NEG = -1e9


def _qkv_kernel(x_ref, wq_ref, wk_ref, wv_ref, o_ref, w_s):
    @pl.when(pl.program_id(0) == 0)
    def _():
        w_s[:, :DIM] = wq_ref[...].astype(jnp.bfloat16)
        w_s[:, DIM:2 * DIM] = wk_ref[...].astype(jnp.bfloat16)
        w_s[:, 2 * DIM:] = wv_ref[...].astype(jnp.bfloat16)

    xb = x_ref[...].astype(jnp.bfloat16)
    o_ref[...] = jnp.dot(xb, w_s[...],
                         preferred_element_type=jnp.float32
                         ).astype(jnp.bfloat16)


def _out_kernel(a_ref, wo_ref, o_ref, w_s):
    @pl.when(pl.program_id(0) == 0)
    def _():
        w_s[...] = wo_ref[...].astype(jnp.bfloat16)

    o_ref[...] = jnp.dot(a_ref[...], w_s[...],
                         preferred_element_type=jnp.float32)


def _rowmeans(x2, n):
    i = jax.lax.broadcasted_iota(jnp.int32, (n, n), 0)
    j = jax.lax.broadcasted_iota(jnp.int32, (n, n), 1)
    ms = jnp.where((i // HDIM) == (j // HDIM), 1.0 / HDIM, 0.0
                   ).astype(jnp.bfloat16)
    return jnp.dot(x2.astype(jnp.bfloat16), ms,
                   preferred_element_type=jnp.float32)


def _attn_kernel(q_ref, k_ref, v_ref, gq_ref, gk_ref, wo_ref, o_ref,
                 kn_ref, vx_ref, cb_ref, as_ref, wo_s):
    p_idx = pl.program_id(0)
    qi = jax.lax.broadcasted_iota(jnp.int32, (ROW, KVW), 0)
    kj = jax.lax.broadcasted_iota(jnp.int32, (ROW, KVW), 1)
    d = (kj // TPB) % BW - qi // TPB
    ok = (d >= -(WW // 2)) & (d <= WW - 1 - WW // 2)
    cb_ref[...] = jnp.where(ok, 0.0, NEG).astype(jnp.float32)

    gq2 = jnp.concatenate([gq_ref[...]] * HPG, axis=1) * SCALE
    gk2 = jnp.concatenate([gk_ref[...]] * HPG, axis=1)

    k = k_ref[...].astype(jnp.float32)
    km = _rowmeans(k * k, CW)
    kn_ref[...] = (k * jax.lax.rsqrt(km + EPS) * gk2).astype(jnp.bfloat16)
    v = v_ref[...]
    one = jnp.ones((SEQ, HDIM), jnp.bfloat16)
    for u in range(HPG):
        vx_ref[:, 2 * u * HDIM:(2 * u + 1) * HDIM] = \
            v[:, u * HDIM:(u + 1) * HDIM]
        vx_ref[:, (2 * u + 1) * HDIM:(2 * u + 2) * HDIM] = one

    for r in range(BH):
        lo = max(r - WH // 2, 0) * ROW
        hi = min(r + WH - WH // 2, BH) * ROW
        w = hi - lo

        q = q_ref[r * ROW:(r + 1) * ROW, :].astype(jnp.float32)
        qm = _rowmeans(q * q, CW)
        qn = (q * jax.lax.rsqrt(qm + EPS) * gq2).astype(jnp.bfloat16)

        outs = []
        for u in range(HPG):
            s = jax.lax.dot_general(
                qn[:, u * HDIM:(u + 1) * HDIM], kn_ref[lo:hi,
                                                       u * HDIM:(u + 1) * HDIM],
                (((1,), (1,)), ((), ())),
                preferred_element_type=jnp.float32)
            e = jnp.exp(s + cb_ref[:, :w]).astype(jnp.bfloat16)
            pv = jnp.dot(e, vx_ref[lo:hi, 2 * u * HDIM:(2 * u + 2) * HDIM],
                         preferred_element_type=jnp.float32)
            outs.append((pv[:, :HDIM] * (1.0 / pv[:, HDIM:HDIM + 1])
                         ).astype(jnp.bfloat16))
        as_ref[r * ROW:(r + 1) * ROW, pl.ds(p_idx * CW, CW)] = \
            jnp.concatenate(outs, axis=1)

    @pl.when(p_idx == NPAIR - 1)
    def _():
        wo_s[...] = wo_ref[...].astype(jnp.bfloat16)
        o_ref[...] = jnp.dot(as_ref[...], wo_s[...],
                             preferred_element_type=jnp.float32)


def kernel(x, Wq, Wk, Wv, Wo, gq, gk):
    B = x.shape[0]
    x2 = x.reshape(SEQ, DIM)
    gq1 = gq.reshape(1, HDIM)
    gk1 = gk.reshape(1, HDIM)

    qkv = pl.pallas_call(
        _qkv_kernel,
        grid=(SEQ // ROW,),
        in_specs=[
            pl.BlockSpec((ROW, DIM), lambda i: (i, 0)),
            pl.BlockSpec((DIM, DIM), lambda i: (0, 0)),
            pl.BlockSpec((DIM, DIM), lambda i: (0, 0)),
            pl.BlockSpec((DIM, DIM), lambda i: (0, 0)),
        ],
        out_specs=pl.BlockSpec((ROW, 3 * DIM), lambda i: (i, 0)),
        out_shape=jax.ShapeDtypeStruct((SEQ, 3 * DIM), jnp.bfloat16),
        scratch_shapes=[pltpu.VMEM((DIM, 3 * DIM), jnp.bfloat16)],
        compiler_params=pltpu.CompilerParams(
            dimension_semantics=("arbitrary",)),
    )(x2, Wq, Wk, Wv)

    out = pl.pallas_call(
        _attn_kernel,
        grid=(NPAIR,),
        in_specs=[
            pl.BlockSpec((SEQ, CW), lambda p: (0, p)),
            pl.BlockSpec((SEQ, CW), lambda p: (0, NPAIR + p)),
            pl.BlockSpec((SEQ, CW), lambda p: (0, 2 * NPAIR + p)),
            pl.BlockSpec((1, HDIM), lambda p: (0, 0)),
            pl.BlockSpec((1, HDIM), lambda p: (0, 0)),
            pl.BlockSpec((DIM, DIM), lambda p: (0, 0)),
        ],
        out_specs=pl.BlockSpec((SEQ, DIM), lambda p: (0, 0)),
        out_shape=jax.ShapeDtypeStruct((SEQ, DIM), jnp.float32),
        scratch_shapes=[
            pltpu.VMEM((SEQ, CW), jnp.bfloat16),
            pltpu.VMEM((SEQ, 2 * CW), jnp.bfloat16),
            pltpu.VMEM((ROW, KVW), jnp.float32),
            pltpu.VMEM((SEQ, DIM), jnp.bfloat16),
            pltpu.VMEM((DIM, DIM), jnp.bfloat16),
        ],
        compiler_params=pltpu.CompilerParams(
            dimension_semantics=("arbitrary",)),
    )(qkv, qkv, qkv, gq1, gk1, Wo)

    return out.reshape(B, SEQ, DIM)


# speedup vs baseline: 1.1109x; 1.0562x over previous
import jax
import jax.numpy as jnp
from jax.experimental import pallas as pl
from jax.experimental.pallas import tpu as pltpu

DIM = 1024
HEADS = 16
HDIM = DIM // HEADS
SEQ = 2048
BH, BW = 8, 8
WH, WW = 6, 6
EPS = 1e-6
TPB = SEQ // (BH * BW)
ROW = BW * TPB
KVW = WH * ROW
SCALE = 1.0 / (HDIM ** 0.5)
HPG = 4
CW = HPG * HDIM
NPAIR = HEADS // HPG
NEG = -1e9


def _qkv_kernel(x_ref, wq_ref, wk_ref, wv_ref, o_ref, w_s):
    @pl.when(pl.program_id(0) == 0)
    def _():
        w_s[:, :DIM] = wq_ref[...].astype(jnp.bfloat16)
        w_s[:, DIM:2 * DIM] = wk_ref[...].astype(jnp.bfloat16)
        w_s[:, 2 * DIM:] = wv_ref[...].astype(jnp.bfloat16)

    xb = x_ref[...].astype(jnp.bfloat16)
    o_ref[...] = jnp.dot(xb, w_s[...],
                         preferred_element_type=jnp.float32
                         ).astype(jnp.bfloat16)


def _out_kernel(a_ref, wo_ref, o_ref, w_s):
    @pl.when(pl.program_id(0) == 0)
    def _():
        w_s[...] = wo_ref[...].astype(jnp.bfloat16)

    o_ref[...] = jnp.dot(a_ref[...], w_s[...],
                         preferred_element_type=jnp.float32)


def _rowmeans(x2, n):
    i = jax.lax.broadcasted_iota(jnp.int32, (n, n), 0)
    j = jax.lax.broadcasted_iota(jnp.int32, (n, n), 1)
    ms = jnp.where((i // HDIM) == (j // HDIM), 1.0 / HDIM, 0.0
                   ).astype(jnp.bfloat16)
    return jnp.dot(x2.astype(jnp.bfloat16), ms,
                   preferred_element_type=jnp.float32)


def _attn_kernel(x_ref, wq_ref, wk_ref, wv_ref, gq_ref, gk_ref, wo_ref,
                 o_ref, xb_ref, qkv_ref, kn_ref, vx_ref, cb_ref, as_ref,
                 wo_s):
    p_idx = pl.program_id(0)

    @pl.when(p_idx == 0)
    def _():
        xb_ref[...] = x_ref[...].astype(jnp.bfloat16)

    xb = xb_ref[...]
    qkv_ref[:, :CW] = jnp.dot(xb, wq_ref[...].astype(jnp.bfloat16),
                              preferred_element_type=jnp.float32
                              ).astype(jnp.bfloat16)
    qkv_ref[:, CW:2 * CW] = jnp.dot(xb, wk_ref[...].astype(jnp.bfloat16),
                                    preferred_element_type=jnp.float32
                                    ).astype(jnp.bfloat16)
    qkv_ref[:, 2 * CW:] = jnp.dot(xb, wv_ref[...].astype(jnp.bfloat16),
                                  preferred_element_type=jnp.float32
                                  ).astype(jnp.bfloat16)
    qi = jax.lax.broadcasted_iota(jnp.int32, (ROW, KVW), 0)
    kj = jax.lax.broadcasted_iota(jnp.int32, (ROW, KVW), 1)
    d = (kj // TPB) % BW - qi // TPB
    ok = (d >= -(WW // 2)) & (d <= WW - 1 - WW // 2)
    cb_ref[...] = jnp.where(ok, 0.0, NEG).astype(jnp.float32)

    gq2 = jnp.concatenate([gq_ref[...]] * HPG, axis=1) * SCALE
    gk2 = jnp.concatenate([gk_ref[...]] * HPG, axis=1)

    k = qkv_ref[:, CW:2 * CW].astype(jnp.float32)
    km = _rowmeans(k * k, CW)
    kn_ref[...] = (k * jax.lax.rsqrt(km + EPS) * gk2).astype(jnp.bfloat16)
    v = qkv_ref[:, 2 * CW:]
    one = jnp.ones((SEQ, HDIM), jnp.bfloat16)
    for u in range(HPG):
        vx_ref[:, 2 * u * HDIM:(2 * u + 1) * HDIM] = \
            v[:, u * HDIM:(u + 1) * HDIM]
        vx_ref[:, (2 * u + 1) * HDIM:(2 * u + 2) * HDIM] = one

    for r in range(BH):
        lo = max(r - WH // 2, 0) * ROW
        hi = min(r + WH - WH // 2, BH) * ROW
        w = hi - lo

        q = qkv_ref[r * ROW:(r + 1) * ROW, :CW].astype(jnp.float32)
        qm = _rowmeans(q * q, CW)
        qn = (q * jax.lax.rsqrt(qm + EPS) * gq2).astype(jnp.bfloat16)

        outs = []
        for u in range(HPG):
            s = jax.lax.dot_general(
                qn[:, u * HDIM:(u + 1) * HDIM], kn_ref[lo:hi,
                                                       u * HDIM:(u + 1) * HDIM],
                (((1,), (1,)), ((), ())),
                preferred_element_type=jnp.float32)
            e = jnp.exp(s + cb_ref[:, :w]).astype(jnp.bfloat16)
            pv = jnp.dot(e, vx_ref[lo:hi, 2 * u * HDIM:(2 * u + 2) * HDIM],
                         preferred_element_type=jnp.float32)
            outs.append((pv[:, :HDIM] * (1.0 / pv[:, HDIM:HDIM + 1])
                         ).astype(jnp.bfloat16))
        as_ref[r * ROW:(r + 1) * ROW, pl.ds(p_idx * CW, CW)] = \
            jnp.concatenate(outs, axis=1)

    @pl.when(p_idx == NPAIR - 1)
    def _():
        wo_s[...] = wo_ref[...].astype(jnp.bfloat16)
        o_ref[...] = jnp.dot(as_ref[...], wo_s[...],
                             preferred_element_type=jnp.float32)


def kernel(x, Wq, Wk, Wv, Wo, gq, gk):
    B = x.shape[0]
    x2 = x.reshape(SEQ, DIM)
    gq1 = gq.reshape(1, HDIM)
    gk1 = gk.reshape(1, HDIM)

    out = pl.pallas_call(
        _attn_kernel,
        grid=(NPAIR,),
        in_specs=[
            pl.BlockSpec((SEQ, DIM), lambda p: (0, 0)),
            pl.BlockSpec((DIM, CW), lambda p: (0, p)),
            pl.BlockSpec((DIM, CW), lambda p: (0, p)),
            pl.BlockSpec((DIM, CW), lambda p: (0, p)),
            pl.BlockSpec((1, HDIM), lambda p: (0, 0)),
            pl.BlockSpec((1, HDIM), lambda p: (0, 0)),
            pl.BlockSpec((DIM, DIM), lambda p: (0, 0)),
        ],
        out_specs=pl.BlockSpec((SEQ, DIM), lambda p: (0, 0)),
        out_shape=jax.ShapeDtypeStruct((SEQ, DIM), jnp.float32),
        scratch_shapes=[
            pltpu.VMEM((SEQ, DIM), jnp.bfloat16),
            pltpu.VMEM((SEQ, 3 * CW), jnp.bfloat16),
            pltpu.VMEM((SEQ, CW), jnp.bfloat16),
            pltpu.VMEM((SEQ, 2 * CW), jnp.bfloat16),
            pltpu.VMEM((ROW, KVW), jnp.float32),
            pltpu.VMEM((SEQ, DIM), jnp.bfloat16),
            pltpu.VMEM((DIM, DIM), jnp.bfloat16),
        ],
        compiler_params=pltpu.CompilerParams(
            dimension_semantics=("arbitrary",)),
    )(x2, Wq, Wk, Wv, gq1, gk1, Wo)

    return out.reshape(B, SEQ, DIM)


# speedup vs baseline: 1.1895x; 1.0707x over previous
import jax
import jax.numpy as jnp
from jax.experimental import pallas as pl
from jax.experimental.pallas import tpu as pltpu

DIM = 1024
HEADS = 16
HDIM = DIM // HEADS
SEQ = 2048
BH, BW = 8, 8
WH, WW = 6, 6
EPS = 1e-6
TPB = SEQ // (BH * BW)
ROW = BW * TPB
KVW = WH * ROW
SCALE = 1.0 / (HDIM ** 0.5)
HPG = 4
CW = HPG * HDIM
NPAIR = HEADS // HPG
NEG = -1e9


def _qkv_kernel(x_ref, wq_ref, wk_ref, wv_ref, o_ref, w_s):
    @pl.when(pl.program_id(0) == 0)
    def _():
        w_s[:, :DIM] = wq_ref[...].astype(jnp.bfloat16)
        w_s[:, DIM:2 * DIM] = wk_ref[...].astype(jnp.bfloat16)
        w_s[:, 2 * DIM:] = wv_ref[...].astype(jnp.bfloat16)

    xb = x_ref[...].astype(jnp.bfloat16)
    o_ref[...] = jnp.dot(xb, w_s[...],
                         preferred_element_type=jnp.float32
                         ).astype(jnp.bfloat16)


def _out_kernel(a_ref, wo_ref, o_ref, w_s):
    @pl.when(pl.program_id(0) == 0)
    def _():
        w_s[...] = wo_ref[...].astype(jnp.bfloat16)

    o_ref[...] = jnp.dot(a_ref[...], w_s[...],
                         preferred_element_type=jnp.float32)


def _rowmeans(x2, n):
    i = jax.lax.broadcasted_iota(jnp.int32, (n, n), 0)
    j = jax.lax.broadcasted_iota(jnp.int32, (n, n), 1)
    ms = jnp.where((i // HDIM) == (j // HDIM), 1.0 / HDIM, 0.0
                   ).astype(jnp.bfloat16)
    return jnp.dot(x2.astype(jnp.bfloat16), ms,
                   preferred_element_type=jnp.float32)


def _attn_kernel(x_ref, wq_ref, wk_ref, wv_ref, gq_ref, gk_ref, wo_ref,
                 o_ref, xb_ref, qkv_ref, kn_ref, vx_ref, cb_ref, as_ref,
                 wo_s):
    p_idx = pl.program_id(0)

    @pl.when(p_idx == 0)
    def _():
        xb_ref[...] = x_ref[...].astype(jnp.bfloat16)

    xb = xb_ref[...]
    qkv_ref[:, :CW] = jnp.dot(xb, wq_ref[...].astype(jnp.bfloat16),
                              preferred_element_type=jnp.float32
                              ).astype(jnp.bfloat16)
    qkv_ref[:, CW:2 * CW] = jnp.dot(xb, wk_ref[...].astype(jnp.bfloat16),
                                    preferred_element_type=jnp.float32
                                    ).astype(jnp.bfloat16)
    qkv_ref[:, 2 * CW:] = jnp.dot(xb, wv_ref[...].astype(jnp.bfloat16),
                                  preferred_element_type=jnp.float32
                                  ).astype(jnp.bfloat16)
    qi = jax.lax.broadcasted_iota(jnp.int32, (ROW, KVW), 0)
    kj = jax.lax.broadcasted_iota(jnp.int32, (ROW, KVW), 1)
    d = (kj // TPB) % BW - qi // TPB
    ok = (d >= -(WW // 2)) & (d <= WW - 1 - WW // 2)
    cb_ref[...] = jnp.where(ok, 0.0, NEG).astype(jnp.float32)

    gq2 = jnp.concatenate([gq_ref[...]] * HPG, axis=1) * SCALE
    gk2 = jnp.concatenate([gk_ref[...]] * HPG, axis=1)

    k = qkv_ref[:, CW:2 * CW].astype(jnp.float32)
    km = _rowmeans(k * k, CW)
    kn_ref[...] = (k * jax.lax.rsqrt(km + EPS) * gk2).astype(jnp.bfloat16)
    v = qkv_ref[:, 2 * CW:]
    one = jnp.ones((SEQ, HDIM), jnp.bfloat16)
    for u in range(HPG):
        vx_ref[:, 2 * u * HDIM:(2 * u + 1) * HDIM] = \
            v[:, u * HDIM:(u + 1) * HDIM]
        vx_ref[:, (2 * u + 1) * HDIM:(2 * u + 2) * HDIM] = one

    for r in range(BH):
        lo = max(r - WH // 2, 0) * ROW
        hi = min(r + WH - WH // 2, BH) * ROW
        w = hi - lo

        q = qkv_ref[r * ROW:(r + 1) * ROW, :CW].astype(jnp.float32)
        qm = _rowmeans(q * q, CW)
        qn = q * jax.lax.rsqrt(qm + EPS) * gq2

        outs = []
        for u in range(HPG):
            s = jax.lax.dot_general(
                qn[:, u * HDIM:(u + 1) * HDIM], kn_ref[lo:hi,
                                                       u * HDIM:(u + 1) * HDIM],
                (((1,), (1,)), ((), ())),
                preferred_element_type=jnp.float32)
            e = jnp.exp(s + cb_ref[:, :w])
            pv = jnp.dot(e, vx_ref[lo:hi, 2 * u * HDIM:(2 * u + 2) * HDIM],
                         preferred_element_type=jnp.float32)
            outs.append((pv[:, :HDIM] * (1.0 / pv[:, HDIM:HDIM + 1])
                         ).astype(jnp.bfloat16))
        as_ref[r * ROW:(r + 1) * ROW, pl.ds(p_idx * CW, CW)] = \
            jnp.concatenate(outs, axis=1)

    @pl.when(p_idx == NPAIR - 1)
    def _():
        wo_s[...] = wo_ref[...].astype(jnp.bfloat16)
        o_ref[...] = jnp.dot(as_ref[...], wo_s[...],
                             preferred_element_type=jnp.float32)


def kernel(x, Wq, Wk, Wv, Wo, gq, gk):
    B = x.shape[0]
    x2 = x.reshape(SEQ, DIM)
    gq1 = gq.reshape(1, HDIM)
    gk1 = gk.reshape(1, HDIM)

    out = pl.pallas_call(
        _attn_kernel,
        grid=(NPAIR,),
        in_specs=[
            pl.BlockSpec((SEQ, DIM), lambda p: (0, 0)),
            pl.BlockSpec((DIM, CW), lambda p: (0, p)),
            pl.BlockSpec((DIM, CW), lambda p: (0, p)),
            pl.BlockSpec((DIM, CW), lambda p: (0, p)),
            pl.BlockSpec((1, HDIM), lambda p: (0, 0)),
            pl.BlockSpec((1, HDIM), lambda p: (0, 0)),
            pl.BlockSpec((DIM, DIM), lambda p: (0, 0)),
        ],
        out_specs=pl.BlockSpec((SEQ, DIM), lambda p: (0, 0)),
        out_shape=jax.ShapeDtypeStruct((SEQ, DIM), jnp.float32),
        scratch_shapes=[
            pltpu.VMEM((SEQ, DIM), jnp.bfloat16),
            pltpu.VMEM((SEQ, 3 * CW), jnp.bfloat16),
            pltpu.VMEM((SEQ, CW), jnp.bfloat16),
            pltpu.VMEM((SEQ, 2 * CW), jnp.bfloat16),
            pltpu.VMEM((ROW, KVW), jnp.float32),
            pltpu.VMEM((SEQ, DIM), jnp.bfloat16),
            pltpu.VMEM((DIM, DIM), jnp.bfloat16),
        ],
        compiler_params=pltpu.CompilerParams(
            dimension_semantics=("arbitrary",)),
    )(x2, Wq, Wk, Wv, gq1, gk1, Wo)

    return out.reshape(B, SEQ, DIM)
